# baseline (speedup 1.0000x reference)
import functools

import jax
import jax.numpy as jnp
from jax import lax
from jax.experimental import pallas as pl
from jax.experimental.pallas import tpu as pltpu
from jax.experimental.pallas import tpu_sc as plsc

_D = 1024
_NC = 2
_NS = 16
_NW = _NC * _NS
_CH = 32


def _sc_kernel(seq2, mask_i, mrow, w0row, n_rows):
    rpw = n_rows // _NW
    n_pairs = rpw // (2 * _CH)
    row_b = 4 * _D
    out_b = 8 * _D
    mesh = plsc.VectorSubcoreMesh(core_axis_name="c", subcore_axis_name="s")

    @functools.partial(
        pl.kernel,
        mesh=mesh,
        out_type=jax.ShapeDtypeStruct((n_rows, 2 * _D), jnp.float32),
        scratch_types=[
            pltpu.VMEM((rpw,), jnp.int32),
            pltpu.VMEM((1, 2 * _D), jnp.float32),
            pltpu.VMEM((1, _D), jnp.float32),
            pltpu.VMEM((_CH, _D), jnp.float32),
            pltpu.VMEM((_CH, _D), jnp.float32),
            pltpu.SemaphoreType.DMA,
            pltpu.SemaphoreType.DMA,
            pltpu.SemaphoreType.DMA,
            pltpu.SemaphoreType.DMA,
        ],
    )
    def body(seq_hbm, mask_hbm, mrow_hbm, w0_hbm, out_hbm,
             midx_v, mrow_v, w0_v, ring0, ring1,
             rsem0, rsem1, wsem0, wsem1):
        cid = lax.axis_index("c")
        sid = lax.axis_index("s")
        wid = sid * _NC + cid
        base = wid * rpw

        pltpu.sync_copy(mrow_hbm, mrow_v)
        pltpu.sync_copy(w0_hbm, w0_v)
        pltpu.sync_copy(mask_hbm.at[pl.ds(base, rpw)], midx_v)

        def bits(chunk):
            ms = []
            for gg in range(_CH // 16):
                v16 = midx_v[pl.ds(chunk * _CH + gg * 16, 16)]
                for l in range(16):
                    ms.append(v16[l])
            return ms

        def fire_reads(chunk, ring, rsem):
            pltpu.async_copy(
                seq_hbm.at[pl.ds(base + chunk * _CH, _CH), pl.ds(0, _D)],
                ring, rsem)

        def fire_writes(ms, chunk, ring, wsem):
            for j, m in enumerate(ms):
                row = base + chunk * _CH + j

                @pl.when(m == 1)
                def _():
                    pltpu.async_copy(
                        mrow_v,
                        out_hbm.at[pl.ds(row, 1), pl.ds(0, 2 * _D)],
                        wsem)

                @pl.when(m == 0)
                def _():
                    pltpu.async_copy(
                        ring.at[pl.ds(j, 1), pl.ds(0, _D)],
                        out_hbm.at[pl.ds(row, 1), pl.ds(0, _D)],
                        wsem)
                    pltpu.async_copy(
                        w0_v,
                        out_hbm.at[pl.ds(row, 1), pl.ds(_D, _D)],
                        wsem)

        def drain(wsem, ring):
            pltpu.make_async_copy(
                out_hbm.at[pl.ds(base, _CH), pl.ds(0, _D)], ring, wsem).wait()
            pltpu.make_async_copy(
                out_hbm.at[pl.ds(base, _CH), pl.ds(0, _D)], ring, wsem).wait()

        def wait_reads(rsem, ring):
            pltpu.make_async_copy(
                out_hbm.at[pl.ds(base, _CH), pl.ds(0, _D)], ring, rsem).wait()

        def pair(i, carry):
            a = 2 * i
            b = 2 * i + 1

            @pl.when(i >= 1)
            def _():
                drain(wsem0, ring0)

            fire_reads(a, ring0, rsem0)

            @pl.when(i >= 1)
            def _():
                drain(wsem1, ring1)

            fire_reads(b, ring1, rsem1)

            wait_reads(rsem0, ring0)
            fire_writes(bits(a), a, ring0, wsem0)
            wait_reads(rsem1, ring1)
            fire_writes(bits(b), b, ring1, wsem1)
            return carry

        lax.fori_loop(0, n_pairs, pair, 0)
        drain(wsem0, ring0)
        drain(wsem1, ring1)

    return body(seq2, mask_i, mrow, w0row)


def kernel(seq, mask, emb_weight):
    B, S, D = seq.shape
    N = B * S
    seq2 = seq.reshape(N, D)
    mask_i = mask.astype(jnp.int32).reshape(N)

    zrow = jnp.zeros((1, D), jnp.float32)
    mrow = jnp.concatenate([zrow, emb_weight[1:2, :]], axis=1)
    w0row = emb_weight[0:1, :]

    out = _sc_kernel(seq2, mask_i, mrow, w0row, N)
    return out.reshape(B, S, 2 * D)


# speedup vs baseline: 1.2025x; 1.2025x over previous
import functools

import jax
import jax.numpy as jnp
from jax import lax
from jax.experimental import pallas as pl
from jax.experimental.pallas import tpu as pltpu
from jax.experimental.pallas import tpu_sc as plsc

_D = 1024
_ROWS = 2048
_NC = 2
_NS = 16
_NW = _NC * _NS
_CHUNK = 32


def _sc_phase(mask_i, emb_weight, n_rows):
    rpw = n_rows // _NW
    n_groups = rpw // 16
    mesh = plsc.VectorSubcoreMesh(core_axis_name="c", subcore_axis_name="s")

    @functools.partial(
        pl.kernel,
        mesh=mesh,
        out_type=jax.ShapeDtypeStruct((n_rows, 2 * _D), jnp.float32),
        scratch_types=[
            pltpu.VMEM((rpw,), jnp.int32),
            pltpu.VMEM((2, _D), jnp.float32),
            pltpu.VMEM((_CHUNK, _D), jnp.float32),
            pltpu.SemaphoreType.DMA,
        ],
    )
    def body(mask_hbm, emb_hbm, out_hbm, midx_v, table_v, drain_v, wsem):
        cid = lax.axis_index("c")
        sid = lax.axis_index("s")
        wid = sid * _NC + cid
        base = wid * rpw

        pltpu.sync_copy(emb_hbm, table_v)
        pltpu.sync_copy(mask_hbm.at[pl.ds(base, rpw)], midx_v)

        def group(g, carry):
            v16 = midx_v[pl.ds(g * 16, 16)]
            for l in range(16):
                m_l = v16[l]
                row = base + g * 16 + l
                pltpu.async_copy(
                    table_v.at[pl.ds(m_l, 1)],
                    out_hbm.at[pl.ds(row, 1), pl.ds(_D, _D)],
                    wsem)
            return carry

        lax.fori_loop(0, n_groups, group, 0)

        def drain(k, carry):
            pltpu.make_async_copy(
                out_hbm.at[pl.ds(base, _CHUNK), pl.ds(_D, _D)],
                drain_v, wsem).wait()
            return carry

        lax.fori_loop(0, rpw // _CHUNK, drain, 0)

    return body(mask_i, emb_weight)


def _tc_body(mask_ref, seq_ref, buf_ref, out_ref):
    m = mask_ref[0]
    keep = (m.reshape(_ROWS, 1) == 0)
    out_ref[...] = jnp.where(keep, seq_ref[...], 0.0)


def kernel(seq, mask, emb_weight):
    B, S, D = seq.shape
    N = B * S
    G = N // _ROWS
    seq2 = seq.reshape(N, D)
    mask_i = mask.astype(jnp.int32)
    mask3 = mask_i.reshape(G, 1, _ROWS)

    buf = _sc_phase(mask_i.reshape(N), emb_weight, N)

    out = pl.pallas_call(
        _tc_body,
        grid=(G,),
        in_specs=[
            pl.BlockSpec((1, 1, _ROWS), lambda i: (i, 0, 0)),
            pl.BlockSpec((_ROWS, D), lambda i: (i, 0)),
            pl.BlockSpec((8, 128), lambda i: (0, 0)),
        ],
        out_specs=pl.BlockSpec((_ROWS, D), lambda i: (i, 0)),
        out_shape=jax.ShapeDtypeStruct((N, 2 * D), jnp.float32),
        input_output_aliases={2: 0},
    )(mask3, seq2, buf)
    return out.reshape(B, S, 2 * D)
